# baseline (speedup 1.0000x reference)
import functools

import jax
import jax.numpy as jnp
from jax.experimental import pallas as pl
from jax.experimental.pallas import tpu as pltpu

TOPK = 8
E = 64
BM = 2048
BK = 2048
NK = 4096 // BK
NEG = -3.0e38


def _router_kernel(x_ref, wt_ref, b_ref, w_out_ref, i_out_ref, acc_ref):
    k = pl.program_id(1)
    part = jnp.dot(x_ref[...], wt_ref[...], preferred_element_type=jnp.float32)

    @pl.when(k == 0)
    def _():
        acc_ref[...] = part

    @pl.when(k != 0)
    def _():
        acc_ref[...] += part

    @pl.when(k == NK - 1)
    def _():
        _router_select(acc_ref[...], b_ref, w_out_ref, i_out_ref)


def _router_select(logits, b_ref, w_out_ref, i_out_ref):
    lt = logits.T
    scores = jax.nn.sigmoid(lt)
    biased = scores + b_ref[...]
    rows = jax.lax.broadcasted_iota(jnp.int32, biased.shape, 0).astype(jnp.float32)

    idx_parts = []
    w_parts = []
    cur = biased
    for _ in range(TOPK):
        m = jnp.max(cur, axis=0, keepdims=True)
        is_max = cur == m
        idx_k = jnp.min(jnp.where(is_max, rows, float(E)), axis=0, keepdims=True)
        sel = rows == idx_k
        w_k = jnp.sum(jnp.where(sel, scores, 0.0), axis=0, keepdims=True)
        idx_parts.append(idx_k)
        w_parts.append(w_k)
        cur = jnp.where(sel, NEG, cur)

    w = jnp.concatenate(w_parts, axis=0)
    idx = jnp.concatenate(idx_parts, axis=0)
    w = w / (jnp.sum(w, axis=0, keepdims=True) + 1e-20)
    w_out_ref[...] = w
    i_out_ref[...] = idx.astype(jnp.int32)


@functools.partial(jax.jit, static_argnames=())
def kernel(x, weight, bias):
    t = x.shape[0]
    wt = weight.T
    bt = bias.reshape(E, 1)
    grid = (t // BM, NK)
    w_t, idx_t = pl.pallas_call(
        _router_kernel,
        grid=grid,
        in_specs=[
            pl.BlockSpec((BM, BK), lambda i, k: (i, k)),
            pl.BlockSpec((BK, E), lambda i, k: (k, 0)),
            pl.BlockSpec((E, 1), lambda i, k: (0, 0)),
        ],
        out_specs=[
            pl.BlockSpec((TOPK, BM), lambda i, k: (0, i)),
            pl.BlockSpec((TOPK, BM), lambda i, k: (0, i)),
        ],
        out_shape=[
            jax.ShapeDtypeStruct((TOPK, t), jnp.float32),
            jax.ShapeDtypeStruct((TOPK, t), jnp.int32),
        ],
        scratch_shapes=[pltpu.VMEM((BM, E), jnp.float32)],
    )(x, wt, bt)
    return w_t.T, idx_t.T
